# baseline (speedup 1.0000x reference)
import jax
import jax.numpy as jnp
from jax.experimental import pallas as pl
from jax.experimental.pallas import tpu as pltpu

_T = 512
_H = 768
_DFF = 768
_E = 64
_K = 2
_EPS = 1e-6

_B = 32
_G = 96
_PAD = _B * _G


def _router_sort_kernel(x_ref, rw_ref, gw_ref, gb_ref,
                        tbf_ref, stok_ref, scoef_ref, be_ref):
    x = x_ref[...]
    var = jnp.mean(x * x, axis=1, keepdims=True)
    t = x * jax.lax.rsqrt(var + _EPS) * rw_ref[...]
    tbf_ref[...] = t.astype(jnp.bfloat16)

    logits = jax.lax.dot_general(
        t, gw_ref[...], (((1,), (1,)), ((), ())),
        preferred_element_type=jnp.float32) + gb_ref[...]
    iota_e = jax.lax.broadcasted_iota(jnp.int32, (_T, _E), 1)
    m1 = jnp.max(logits, axis=1, keepdims=True)
    i1 = jnp.min(jnp.where(logits == m1, iota_e, _E), axis=1, keepdims=True)
    l2 = jnp.where(iota_e == i1, -jnp.inf, logits)
    m2 = jnp.max(l2, axis=1, keepdims=True)
    i2 = jnp.min(jnp.where(l2 == m2, iota_e, _E), axis=1, keepdims=True)
    a = jnp.exp(m2 - m1)
    w1c = 1.0 / (1.0 + a)
    w2c = a / (1.0 + a)

    oh0 = (iota_e == i1).astype(jnp.float32)
    oh1 = (iota_e == i2).astype(jnp.float32)
    cnt = oh0 + oh1

    tri = (jax.lax.broadcasted_iota(jnp.int32, (_T, _T), 1) <
           jax.lax.broadcasted_iota(jnp.int32, (_T, _T), 0)).astype(jnp.float32)
    csum = jax.lax.dot_general(tri, cnt, (((1,), (0,)), ((), ())),
                               preferred_element_type=jnp.float32,
                               precision=jax.lax.Precision.HIGHEST)

    ones_t = jnp.ones((_T, 1), jnp.float32)
    tot = jax.lax.dot_general(cnt, ones_t, (((0,), (0,)), ((), ())),
                              preferred_element_type=jnp.float32,
                              precision=jax.lax.Precision.HIGHEST)
    nb = jnp.floor((tot + (_B - 1)) / _B)
    lower = (jax.lax.broadcasted_iota(jnp.int32, (_E, _E), 1) <=
             jax.lax.broadcasted_iota(jnp.int32, (_E, _E), 0)).astype(jnp.float32)
    bend = jax.lax.dot_general(lower, nb, (((1,), (0,)), ((), ())),
                               preferred_element_type=jnp.float32,
                               precision=jax.lax.Precision.HIGHEST)
    poff = _B * (bend - nb)

    poff0 = jax.lax.dot_general(oh0, poff, (((1,), (0,)), ((), ())),
                                preferred_element_type=jnp.float32,
                                precision=jax.lax.Precision.HIGHEST)
    poff1 = jax.lax.dot_general(oh1, poff, (((1,), (0,)), ((), ())),
                                preferred_element_type=jnp.float32,
                                precision=jax.lax.Precision.HIGHEST)
    pos0 = poff0 + jnp.sum(oh0 * csum, axis=1, keepdims=True)
    pos1 = poff1 + jnp.sum(oh1 * csum, axis=1, keepdims=True)

    ipad = jax.lax.broadcasted_iota(jnp.int32, (_T, _PAD), 1)
    m0 = (ipad == pos0.astype(jnp.int32)).astype(jnp.float32)
    m1h = (ipad == pos1.astype(jnp.int32)).astype(jnp.float32)
    tok = jax.lax.broadcasted_iota(jnp.int32, (_T, 1), 0).astype(jnp.float32)
    stok_ref[...] = (
        jax.lax.dot_general(m0, tok, (((0,), (0,)), ((), ())),
                            preferred_element_type=jnp.float32,
                            precision=jax.lax.Precision.HIGHEST) +
        jax.lax.dot_general(m1h, tok, (((0,), (0,)), ((), ())),
                            preferred_element_type=jnp.float32,
                            precision=jax.lax.Precision.HIGHEST))
    scoef_ref[...] = (
        jax.lax.dot_general(m0, w1c, (((0,), (0,)), ((), ())),
                            preferred_element_type=jnp.float32,
                            precision=jax.lax.Precision.HIGHEST) +
        jax.lax.dot_general(m1h, w2c, (((0,), (0,)), ((), ())),
                            preferred_element_type=jnp.float32,
                            precision=jax.lax.Precision.HIGHEST))

    ig = jax.lax.broadcasted_iota(jnp.int32, (_E, _G), 1)
    be = jnp.sum((bend.astype(jnp.int32) <= ig).astype(jnp.int32),
                 axis=0, keepdims=True)
    be_ref[...] = jnp.minimum(be, _E - 1)


def _moe_kernel(be_sref, stok_ref, scoef_ref, tbf_ref, x_ref,
                w1_ref, w3_ref, w2_ref, o_ref):
    del be_sref
    g = pl.program_id(0)
    st = stok_ref[...]
    sc = scoef_ref[...]
    iota_t = jax.lax.broadcasted_iota(jnp.int32, (_B, _T), 1)
    p = (iota_t == st.astype(jnp.int32)).astype(jnp.bfloat16)
    tb = jax.lax.dot_general(p, tbf_ref[...], (((1,), (0,)), ((), ())),
                             preferred_element_type=jnp.float32)
    tb = tb.astype(jnp.bfloat16)
    gg = jax.lax.dot_general(tb, w1_ref[0].astype(jnp.bfloat16),
                             (((1,), (1,)), ((), ())),
                             preferred_element_type=jnp.float32)
    uu = jax.lax.dot_general(tb, w3_ref[0].astype(jnp.bfloat16),
                             (((1,), (1,)), ((), ())),
                             preferred_element_type=jnp.float32)
    h = (gg * jax.lax.logistic(gg)) * uu
    y = jax.lax.dot_general(h.astype(jnp.bfloat16),
                            w2_ref[0].astype(jnp.bfloat16),
                            (((1,), (1,)), ((), ())),
                            preferred_element_type=jnp.float32)
    y = y * sc
    contrib = jax.lax.dot_general(p, y.astype(jnp.bfloat16),
                                  (((0,), (0,)), ((), ())),
                                  preferred_element_type=jnp.float32)

    @pl.when(g == 0)
    def _():
        o_ref[...] = x_ref[...] + contrib

    @pl.when(g != 0)
    def _():
        o_ref[...] += contrib


def kernel(x, rms_weight, gate_w, gate_b, w1, w3, w2):
    tbf, stok, scoef, be = pl.pallas_call(
        _router_sort_kernel,
        out_shape=(
            jax.ShapeDtypeStruct((_T, _H), jnp.bfloat16),
            jax.ShapeDtypeStruct((_PAD, 1), jnp.float32),
            jax.ShapeDtypeStruct((_PAD, 1), jnp.float32),
            jax.ShapeDtypeStruct((1, _G), jnp.int32),
        ),
    )(x, rms_weight.reshape(1, _H), gate_w, gate_b.reshape(1, _E))

    grid_spec = pltpu.PrefetchScalarGridSpec(
        num_scalar_prefetch=1,
        grid=(_G,),
        in_specs=[
            pl.BlockSpec((_B, 1), lambda g, be: (g, 0)),
            pl.BlockSpec((_B, 1), lambda g, be: (g, 0)),
            pl.BlockSpec((_T, _H), lambda g, be: (0, 0)),
            pl.BlockSpec((_T, _H), lambda g, be: (0, 0)),
            pl.BlockSpec((1, _DFF, _H), lambda g, be: (be[g], 0, 0)),
            pl.BlockSpec((1, _DFF, _H), lambda g, be: (be[g], 0, 0)),
            pl.BlockSpec((1, _H, _DFF), lambda g, be: (be[g], 0, 0)),
        ],
        out_specs=pl.BlockSpec((_T, _H), lambda g, be: (0, 0)),
    )
    out = pl.pallas_call(
        _moe_kernel,
        grid_spec=grid_spec,
        out_shape=jax.ShapeDtypeStruct((_T, _H), jnp.float32),
        compiler_params=pltpu.CompilerParams(
            dimension_semantics=("arbitrary",)),
    )(be.reshape(_G), stok, scoef, tbf, x, w1, w3, w2)
    return out


# speedup vs baseline: 1.3254x; 1.3254x over previous
import jax
import jax.numpy as jnp
from jax.experimental import pallas as pl
from jax.experimental.pallas import tpu as pltpu

_T = 512
_H = 768
_DFF = 768
_E = 64
_K = 2
_EPS = 1e-6

_B = 32
_G = 96
_PAD = _B * _G


def _router_sort_kernel(x_ref, rw_ref, gw_ref, gb_ref,
                        tbf_ref, stok_ref, scoef_ref, be_ref):
    x = x_ref[...]
    var = jnp.mean(x * x, axis=1, keepdims=True)
    t = x * jax.lax.rsqrt(var + _EPS) * rw_ref[...]
    tbf_ref[...] = t.astype(jnp.bfloat16)

    logits = jax.lax.dot_general(
        t, gw_ref[...], (((1,), (1,)), ((), ())),
        preferred_element_type=jnp.float32) + gb_ref[...]
    iota_e = jax.lax.broadcasted_iota(jnp.int32, (_T, _E), 1)
    m1 = jnp.max(logits, axis=1, keepdims=True)
    i1 = jnp.min(jnp.where(logits == m1, iota_e, _E), axis=1, keepdims=True)
    l2 = jnp.where(iota_e == i1, -jnp.inf, logits)
    m2 = jnp.max(l2, axis=1, keepdims=True)
    i2 = jnp.min(jnp.where(l2 == m2, iota_e, _E), axis=1, keepdims=True)
    a = jnp.exp(m2 - m1)
    w1c = 1.0 / (1.0 + a)
    w2c = a / (1.0 + a)

    oh0 = (iota_e == i1).astype(jnp.bfloat16)
    oh1 = (iota_e == i2).astype(jnp.bfloat16)
    cnt = oh0 + oh1

    tri = (jax.lax.broadcasted_iota(jnp.int32, (_T, _T), 1) <
           jax.lax.broadcasted_iota(jnp.int32, (_T, _T), 0)).astype(jnp.bfloat16)
    csum = jax.lax.dot_general(tri, cnt, (((1,), (0,)), ((), ())),
                               preferred_element_type=jnp.float32)

    ones_t = jnp.ones((_T, 1), jnp.bfloat16)
    tot = jax.lax.dot_general(cnt, ones_t, (((0,), (0,)), ((), ())),
                              preferred_element_type=jnp.float32)
    nb = jnp.floor((tot + (_B - 1)) / _B)
    lower = (jax.lax.broadcasted_iota(jnp.int32, (_E, _E), 1) <=
             jax.lax.broadcasted_iota(jnp.int32, (_E, _E), 0)).astype(jnp.bfloat16)
    bend = jax.lax.dot_general(lower, nb.astype(jnp.bfloat16),
                               (((1,), (0,)), ((), ())),
                               preferred_element_type=jnp.float32)
    bstart = bend - nb

    bs_bf = bstart.astype(jnp.bfloat16)
    poff0 = _B * jax.lax.dot_general(oh0, bs_bf, (((1,), (0,)), ((), ())),
                                     preferred_element_type=jnp.float32)
    poff1 = _B * jax.lax.dot_general(oh1, bs_bf, (((1,), (0,)), ((), ())),
                                     preferred_element_type=jnp.float32)
    ohf0 = oh0.astype(jnp.float32)
    ohf1 = oh1.astype(jnp.float32)
    pos0 = poff0 + jnp.sum(ohf0 * csum, axis=1, keepdims=True)
    pos1 = poff1 + jnp.sum(ohf1 * csum, axis=1, keepdims=True)

    ipad = jax.lax.broadcasted_iota(jnp.int32, (_T, _PAD), 1)
    m0 = (ipad == pos0.astype(jnp.int32)).astype(jnp.bfloat16)
    m1h = (ipad == pos1.astype(jnp.int32)).astype(jnp.bfloat16)
    ms = m0 + m1h
    tok = jax.lax.broadcasted_iota(jnp.int32, (_T, 1), 0)
    tok_lo = (tok % 256).astype(jnp.bfloat16)
    tok_hi = (tok // 256).astype(jnp.bfloat16)
    stok_ref[...] = (
        jax.lax.dot_general(ms, tok_lo, (((0,), (0,)), ((), ())),
                            preferred_element_type=jnp.float32) +
        256.0 * jax.lax.dot_general(ms, tok_hi, (((0,), (0,)), ((), ())),
                                    preferred_element_type=jnp.float32))
    scoef_ref[...] = (
        jax.lax.dot_general(m0, w1c.astype(jnp.bfloat16),
                            (((0,), (0,)), ((), ())),
                            preferred_element_type=jnp.float32) +
        jax.lax.dot_general(m1h, w2c.astype(jnp.bfloat16),
                            (((0,), (0,)), ((), ())),
                            preferred_element_type=jnp.float32))

    ig = jax.lax.broadcasted_iota(jnp.int32, (_E, _G), 1)
    be = jnp.sum((bend.astype(jnp.int32) <= ig).astype(jnp.int32),
                 axis=0, keepdims=True)
    be_ref[:, :_G] = jnp.minimum(be, _E - 1)
    be_ref[:, _G:] = bend[_E - 1:, :].astype(jnp.int32)


def _moe_kernel(be_sref, stok_ref, scoef_ref, tbf_ref, x_ref,
                w1_ref, w3_ref, w2_ref, o_ref):
    g = pl.program_id(0)
    nblk = be_sref[_G]

    @pl.when(g < nblk)
    def _():
        st = stok_ref[...]
        sc = scoef_ref[...]
        iota_t = jax.lax.broadcasted_iota(jnp.int32, (_B, _T), 1)
        p = (iota_t == st.astype(jnp.int32)).astype(jnp.bfloat16)
        tb = jax.lax.dot_general(p, tbf_ref[...], (((1,), (0,)), ((), ())),
                                 preferred_element_type=jnp.float32)
        tb = tb.astype(jnp.bfloat16)
        gg = jax.lax.dot_general(tb, w1_ref[0].astype(jnp.bfloat16),
                                 (((1,), (1,)), ((), ())),
                                 preferred_element_type=jnp.float32)
        uu = jax.lax.dot_general(tb, w3_ref[0].astype(jnp.bfloat16),
                                 (((1,), (1,)), ((), ())),
                                 preferred_element_type=jnp.float32)
        h = (gg * jax.lax.logistic(gg)) * uu
        y = jax.lax.dot_general(h.astype(jnp.bfloat16),
                                w2_ref[0].astype(jnp.bfloat16),
                                (((1,), (1,)), ((), ())),
                                preferred_element_type=jnp.float32)
        y = y * sc
        contrib = jax.lax.dot_general(p, y.astype(jnp.bfloat16),
                                      (((0,), (0,)), ((), ())),
                                      preferred_element_type=jnp.float32)

        @pl.when(g == 0)
        def _():
            o_ref[...] = x_ref[...] + contrib

        @pl.when(g != 0)
        def _():
            o_ref[...] += contrib


def kernel(x, rms_weight, gate_w, gate_b, w1, w3, w2):
    tbf, stok, scoef, be = pl.pallas_call(
        _router_sort_kernel,
        out_shape=(
            jax.ShapeDtypeStruct((_T, _H), jnp.bfloat16),
            jax.ShapeDtypeStruct((_PAD, 1), jnp.float32),
            jax.ShapeDtypeStruct((_PAD, 1), jnp.float32),
            jax.ShapeDtypeStruct((1, _G + 1), jnp.int32),
        ),
    )(x, rms_weight.reshape(1, _H), gate_w, gate_b.reshape(1, _E))

    grid_spec = pltpu.PrefetchScalarGridSpec(
        num_scalar_prefetch=1,
        grid=(_G,),
        in_specs=[
            pl.BlockSpec((_B, 1), lambda g, be: (g, 0)),
            pl.BlockSpec((_B, 1), lambda g, be: (g, 0)),
            pl.BlockSpec((_T, _H), lambda g, be: (0, 0)),
            pl.BlockSpec((_T, _H), lambda g, be: (0, 0)),
            pl.BlockSpec((1, _DFF, _H), lambda g, be: (be[g], 0, 0)),
            pl.BlockSpec((1, _DFF, _H), lambda g, be: (be[g], 0, 0)),
            pl.BlockSpec((1, _H, _DFF), lambda g, be: (be[g], 0, 0)),
        ],
        out_specs=pl.BlockSpec((_T, _H), lambda g, be: (0, 0)),
    )
    out = pl.pallas_call(
        _moe_kernel,
        grid_spec=grid_spec,
        out_shape=jax.ShapeDtypeStruct((_T, _H), jnp.float32),
        compiler_params=pltpu.CompilerParams(
            dimension_semantics=("arbitrary",)),
    )(be.reshape(_G + 1), stok, scoef, tbf, x, w1, w3, w2)
    return out


# speedup vs baseline: 1.4478x; 1.0924x over previous
import jax
import jax.numpy as jnp
from jax.experimental import pallas as pl
from jax.experimental.pallas import tpu as pltpu

_T = 512
_H = 768
_DFF = 768
_E = 64
_K = 2
_EPS = 1e-6

_B = 32
_G = 96
_PAD = _B * _G


def _router_sort_kernel(x_ref, rw_ref, gw_ref, gb_ref,
                        tf_ref, stok_ref, scoef_ref, bsn_ref):
    x = x_ref[...]
    var = jnp.mean(x * x, axis=1, keepdims=True)
    t = x * jax.lax.rsqrt(var + _EPS) * rw_ref[...]
    tf_ref[...] = t

    logits = jax.lax.dot_general(
        t, gw_ref[...], (((1,), (1,)), ((), ())),
        preferred_element_type=jnp.float32) + gb_ref[...]
    iota_e = jax.lax.broadcasted_iota(jnp.int32, (_T, _E), 1)
    m1 = jnp.max(logits, axis=1, keepdims=True)
    i1 = jnp.min(jnp.where(logits == m1, iota_e, _E), axis=1, keepdims=True)
    l2 = jnp.where(iota_e == i1, -jnp.inf, logits)
    m2 = jnp.max(l2, axis=1, keepdims=True)
    i2 = jnp.min(jnp.where(l2 == m2, iota_e, _E), axis=1, keepdims=True)
    a = jnp.exp(m2 - m1)
    w1c = 1.0 / (1.0 + a)
    w2c = a / (1.0 + a)

    oh0 = (iota_e == i1).astype(jnp.bfloat16)
    oh1 = (iota_e == i2).astype(jnp.bfloat16)
    cnt = oh0 + oh1

    tri = (jax.lax.broadcasted_iota(jnp.int32, (_T, _T), 1) <
           jax.lax.broadcasted_iota(jnp.int32, (_T, _T), 0)).astype(jnp.bfloat16)
    csum = jax.lax.dot_general(tri, cnt, (((1,), (0,)), ((), ())),
                               preferred_element_type=jnp.float32)

    ones_t = jnp.ones((_T, 1), jnp.bfloat16)
    tot = jax.lax.dot_general(cnt, ones_t, (((0,), (0,)), ((), ())),
                              preferred_element_type=jnp.float32)
    nb = jnp.floor((tot + (_B - 1)) / _B)
    lower = (jax.lax.broadcasted_iota(jnp.int32, (_E, _E), 1) <=
             jax.lax.broadcasted_iota(jnp.int32, (_E, _E), 0)).astype(jnp.bfloat16)
    bend = jax.lax.dot_general(lower, nb.astype(jnp.bfloat16),
                               (((1,), (0,)), ((), ())),
                               preferred_element_type=jnp.float32)
    bstart = bend - nb

    bs_bf = bstart.astype(jnp.bfloat16)
    poff0 = _B * jax.lax.dot_general(oh0, bs_bf, (((1,), (0,)), ((), ())),
                                     preferred_element_type=jnp.float32)
    poff1 = _B * jax.lax.dot_general(oh1, bs_bf, (((1,), (0,)), ((), ())),
                                     preferred_element_type=jnp.float32)
    ohf0 = oh0.astype(jnp.float32)
    ohf1 = oh1.astype(jnp.float32)
    pos0 = poff0 + jnp.sum(ohf0 * csum, axis=1, keepdims=True)
    pos1 = poff1 + jnp.sum(ohf1 * csum, axis=1, keepdims=True)

    ipad = jax.lax.broadcasted_iota(jnp.int32, (_T, _PAD), 1)
    m0 = (ipad == pos0.astype(jnp.int32)).astype(jnp.bfloat16)
    m1h = (ipad == pos1.astype(jnp.int32)).astype(jnp.bfloat16)
    ms = m0 + m1h
    tok = jax.lax.broadcasted_iota(jnp.int32, (_T, 1), 0)
    tok_lo = (tok % 256).astype(jnp.bfloat16)
    tok_hi = (tok // 256).astype(jnp.bfloat16)
    stok_ref[...] = (
        jax.lax.dot_general(ms, tok_lo, (((0,), (0,)), ((), ())),
                            preferred_element_type=jnp.float32) +
        256.0 * jax.lax.dot_general(ms, tok_hi, (((0,), (0,)), ((), ())),
                                    preferred_element_type=jnp.float32))
    scoef_ref[...] = (
        jax.lax.dot_general(m0, w1c.astype(jnp.bfloat16),
                            (((0,), (0,)), ((), ())),
                            preferred_element_type=jnp.float32) +
        jax.lax.dot_general(m1h, w2c.astype(jnp.bfloat16),
                            (((0,), (0,)), ((), ())),
                            preferred_element_type=jnp.float32))

    ones_e = jnp.ones((_E, 1), jnp.bfloat16)
    bs_row = jax.lax.dot_general(ones_e, bstart.astype(jnp.bfloat16),
                                 (((1,), (1,)), ((), ())),
                                 preferred_element_type=jnp.float32)[0:1]
    nb_row = jax.lax.dot_general(ones_e, nb.astype(jnp.bfloat16),
                                 (((1,), (1,)), ((), ())),
                                 preferred_element_type=jnp.float32)[0:1]
    bsn_ref[:, :_E] = bs_row.astype(jnp.int32)
    bsn_ref[:, _E:] = nb_row.astype(jnp.int32)


def _moe_kernel(bsn_sref, stok_ref, scoef_ref, tf_ref, x_ref,
                w1_ref, w3_ref, w2_ref, o_ref):
    e = pl.program_id(0)

    @pl.when(e == 0)
    def _():
        o_ref[...] = x_ref[...]

    bs = bsn_sref[e]
    nbe = bsn_sref[_E + e]

    def body(i, carry):
        off = (bs + i) * _B
        st = stok_ref[pl.ds(off, _B), :]
        sc = scoef_ref[pl.ds(off, _B), :]
        iota_t = jax.lax.broadcasted_iota(jnp.int32, (_B, _T), 1)
        p = (iota_t == st.astype(jnp.int32)).astype(jnp.float32)
        tb = jax.lax.dot_general(p, tf_ref[...], (((1,), (0,)), ((), ())),
                                 preferred_element_type=jnp.float32)
        gg = jax.lax.dot_general(tb, w1_ref[0],
                                 (((1,), (1,)), ((), ())),
                                 preferred_element_type=jnp.float32)
        uu = jax.lax.dot_general(tb, w3_ref[0],
                                 (((1,), (1,)), ((), ())),
                                 preferred_element_type=jnp.float32)
        h = (gg * jax.lax.logistic(gg)) * uu
        y = jax.lax.dot_general(h, w2_ref[0],
                                (((1,), (1,)), ((), ())),
                                preferred_element_type=jnp.float32)
        y = y * sc
        contrib = jax.lax.dot_general(p, y,
                                      (((0,), (0,)), ((), ())),
                                      preferred_element_type=jnp.float32)
        o_ref[...] += contrib
        return carry

    jax.lax.fori_loop(0, nbe, body, 0)


def kernel(x, rms_weight, gate_w, gate_b, w1, w3, w2):
    tf, stok, scoef, bsn = pl.pallas_call(
        _router_sort_kernel,
        out_shape=(
            jax.ShapeDtypeStruct((_T, _H), jnp.float32),
            jax.ShapeDtypeStruct((_PAD, 1), jnp.float32),
            jax.ShapeDtypeStruct((_PAD, 1), jnp.float32),
            jax.ShapeDtypeStruct((1, 2 * _E), jnp.int32),
        ),
    )(x, rms_weight.reshape(1, _H), gate_w, gate_b.reshape(1, _E))

    grid_spec = pltpu.PrefetchScalarGridSpec(
        num_scalar_prefetch=1,
        grid=(_E,),
        in_specs=[
            pl.BlockSpec((_PAD, 1), lambda e, s: (0, 0)),
            pl.BlockSpec((_PAD, 1), lambda e, s: (0, 0)),
            pl.BlockSpec((_T, _H), lambda e, s: (0, 0)),
            pl.BlockSpec((_T, _H), lambda e, s: (0, 0)),
            pl.BlockSpec((1, _DFF, _H), lambda e, s: (e, 0, 0)),
            pl.BlockSpec((1, _DFF, _H), lambda e, s: (e, 0, 0)),
            pl.BlockSpec((1, _H, _DFF), lambda e, s: (e, 0, 0)),
        ],
        out_specs=pl.BlockSpec((_T, _H), lambda e, s: (0, 0)),
    )
    out = pl.pallas_call(
        _moe_kernel,
        grid_spec=grid_spec,
        out_shape=jax.ShapeDtypeStruct((_T, _H), jnp.float32),
        compiler_params=pltpu.CompilerParams(
            dimension_semantics=("arbitrary",)),
    )(bsn.reshape(2 * _E), stok, scoef, tf, x, w1, w3, w2)
    return out


# speedup vs baseline: 1.4838x; 1.0248x over previous
import jax
import jax.numpy as jnp
from jax.experimental import pallas as pl
from jax.experimental.pallas import tpu as pltpu

_T = 512
_H = 768
_DFF = 768
_E = 64
_EPS = 1e-6

_B = 32
_G = 96


def _router_sort_kernel(x_ref, rw_ref, gw_ref, gb_ref,
                        tf_ref, stok_ref, scoef_ref, bsn_ref):
    x = x_ref[...]
    var = jnp.mean(x * x, axis=1, keepdims=True)
    t = x * jax.lax.rsqrt(var + _EPS) * rw_ref[...]
    tf_ref[...] = t

    logits = jax.lax.dot_general(
        t, gw_ref[...], (((1,), (1,)), ((), ())),
        preferred_element_type=jnp.float32) + gb_ref[...]
    iota_e = jax.lax.broadcasted_iota(jnp.int32, (_T, _E), 1)
    m1 = jnp.max(logits, axis=1, keepdims=True)
    i1 = jnp.min(jnp.where(logits == m1, iota_e, _E), axis=1, keepdims=True)
    l2 = jnp.where(iota_e == i1, -jnp.inf, logits)
    m2 = jnp.max(l2, axis=1, keepdims=True)
    i2 = jnp.min(jnp.where(l2 == m2, iota_e, _E), axis=1, keepdims=True)
    a = jnp.exp(m2 - m1)
    w1c = 1.0 / (1.0 + a)
    w2c = a / (1.0 + a)

    oh0 = (iota_e == i1).astype(jnp.float32)
    oh1 = (iota_e == i2).astype(jnp.float32)
    cnt = oh0 + oh1

    tri = (jax.lax.broadcasted_iota(jnp.int32, (_T, _T), 1) <
           jax.lax.broadcasted_iota(jnp.int32, (_T, _T), 0)).astype(jnp.float32)
    csum = jax.lax.dot_general(tri, cnt, (((1,), (0,)), ((), ())),
                               preferred_element_type=jnp.float32)

    ones_t = jnp.ones((_T, 1), jnp.float32)
    tot = jax.lax.dot_general(cnt, ones_t, (((0,), (0,)), ((), ())),
                              preferred_element_type=jnp.float32)
    nb = jnp.floor((tot + (_B - 1)) / _B)
    lower = (jax.lax.broadcasted_iota(jnp.int32, (_E, _E), 1) <=
             jax.lax.broadcasted_iota(jnp.int32, (_E, _E), 0)).astype(jnp.float32)
    bend = jax.lax.dot_general(lower, nb, (((1,), (0,)), ((), ())),
                               preferred_element_type=jnp.float32)
    bstart = bend - nb

    poff0 = _B * jax.lax.dot_general(oh0, bstart, (((1,), (0,)), ((), ())),
                                     preferred_element_type=jnp.float32)
    poff1 = _B * jax.lax.dot_general(oh1, bstart, (((1,), (0,)), ((), ())),
                                     preferred_element_type=jnp.float32)
    pos0 = poff0 + jnp.sum(oh0 * csum, axis=1, keepdims=True)
    pos1 = poff1 + jnp.sum(oh1 * csum, axis=1, keepdims=True)
    q0 = jnp.floor(pos0 * (1.0 / _B))
    r0 = pos0 - _B * q0
    q1 = jnp.floor(pos1 * (1.0 / _B))
    r1 = pos1 - _B * q1

    iota_q = jax.lax.broadcasted_iota(jnp.int32, (_T, _G), 1)
    iota_r = jax.lax.broadcasted_iota(jnp.int32, (_T, _B), 1)
    mq0 = (iota_q == q0.astype(jnp.int32)).astype(jnp.float32)
    mq1 = (iota_q == q1.astype(jnp.int32)).astype(jnp.float32)
    mr0 = (iota_r == r0.astype(jnp.int32)).astype(jnp.float32)
    mr1 = (iota_r == r1.astype(jnp.int32)).astype(jnp.float32)
    tok = jax.lax.broadcasted_iota(jnp.int32, (_T, 1), 0)
    tok_lo = (tok % 256).astype(jnp.float32)
    tok_hi = (tok // 256).astype(jnp.float32)

    def sc2(lhs, rhs):
        return jax.lax.dot_general(lhs, rhs, (((0,), (0,)), ((), ())),
                                   preferred_element_type=jnp.float32)

    stok_ref[...] = (sc2(mq0, mr0 * tok_lo) + 256.0 * sc2(mq0, mr0 * tok_hi) +
                     sc2(mq1, mr1 * tok_lo) + 256.0 * sc2(mq1, mr1 * tok_hi))
    scoef_ref[...] = sc2(mq0, mr0 * w1c) + sc2(mq1, mr1 * w2c)

    ones_e = jnp.ones((_E, 1), jnp.float32)
    bs_row = jax.lax.dot_general(ones_e, bstart, (((1,), (1,)), ((), ())),
                                 preferred_element_type=jnp.float32)[0:1]
    nb_row = jax.lax.dot_general(ones_e, nb, (((1,), (1,)), ((), ())),
                                 preferred_element_type=jnp.float32)[0:1]
    bsn_ref[:, :_E] = bs_row.astype(jnp.int32)
    bsn_ref[:, _E:] = nb_row.astype(jnp.int32)


def _moe_kernel(bsn_sref, stok_ref, scoef_ref, tf_ref, x_ref,
                w1_ref, w3_ref, w2_ref, o_ref):
    e = pl.program_id(0)

    @pl.when(e == 0)
    def _():
        o_ref[...] = x_ref[...]

    bs = bsn_sref[e]
    nbe = bsn_sref[_E + e]
    iota_tb = jax.lax.broadcasted_iota(jnp.int32, (_T, _B), 0)

    def body(i, carry):
        row = pl.ds(bs + i, 1)
        st = stok_ref[row, :]
        sc = scoef_ref[row, :]
        pt = (iota_tb == st.astype(jnp.int32)).astype(jnp.float32)
        tb = jax.lax.dot_general(pt, tf_ref[...], (((0,), (0,)), ((), ())),
                                 preferred_element_type=jnp.float32)
        gg = jax.lax.dot_general(tb, w1_ref[0],
                                 (((1,), (1,)), ((), ())),
                                 preferred_element_type=jnp.float32)
        uu = jax.lax.dot_general(tb, w3_ref[0],
                                 (((1,), (1,)), ((), ())),
                                 preferred_element_type=jnp.float32)
        h = (gg * jax.lax.logistic(gg)) * uu
        y = jax.lax.dot_general(h, w2_ref[0],
                                (((1,), (1,)), ((), ())),
                                preferred_element_type=jnp.float32)
        contrib = jax.lax.dot_general(pt * sc, y, (((1,), (0,)), ((), ())),
                                      preferred_element_type=jnp.float32)
        o_ref[...] += contrib
        return carry

    jax.lax.fori_loop(0, nbe, body, 0)


def kernel(x, rms_weight, gate_w, gate_b, w1, w3, w2):
    tf, stok, scoef, bsn = pl.pallas_call(
        _router_sort_kernel,
        out_shape=(
            jax.ShapeDtypeStruct((_T, _H), jnp.float32),
            jax.ShapeDtypeStruct((_G, _B), jnp.float32),
            jax.ShapeDtypeStruct((_G, _B), jnp.float32),
            jax.ShapeDtypeStruct((1, 2 * _E), jnp.int32),
        ),
    )(x, rms_weight.reshape(1, _H), gate_w, gate_b.reshape(1, _E))

    grid_spec = pltpu.PrefetchScalarGridSpec(
        num_scalar_prefetch=1,
        grid=(_E,),
        in_specs=[
            pl.BlockSpec((_G, _B), lambda e, s: (0, 0)),
            pl.BlockSpec((_G, _B), lambda e, s: (0, 0)),
            pl.BlockSpec((_T, _H), lambda e, s: (0, 0)),
            pl.BlockSpec((_T, _H), lambda e, s: (0, 0)),
            pl.BlockSpec((1, _DFF, _H), lambda e, s: (e, 0, 0)),
            pl.BlockSpec((1, _DFF, _H), lambda e, s: (e, 0, 0)),
            pl.BlockSpec((1, _H, _DFF), lambda e, s: (e, 0, 0)),
        ],
        out_specs=pl.BlockSpec((_T, _H), lambda e, s: (0, 0)),
    )
    out = pl.pallas_call(
        _moe_kernel,
        grid_spec=grid_spec,
        out_shape=jax.ShapeDtypeStruct((_T, _H), jnp.float32),
        compiler_params=pltpu.CompilerParams(
            dimension_semantics=("arbitrary",)),
    )(bsn.reshape(2 * _E), stok, scoef, tf, x, w1, w3, w2)
    return out


# speedup vs baseline: 1.6295x; 1.0982x over previous
import jax
import jax.numpy as jnp
from jax.experimental import pallas as pl
from jax.experimental.pallas import tpu as pltpu

_T = 512
_H = 768
_DFF = 768
_E = 64
_EPS = 1e-6

_B = 32
_G = 96


def _router_sort_kernel(x_ref, rw_ref, gw_ref, gb_ref,
                        tf_ref, stok_ref, scoef_ref, bsn_ref):
    x = x_ref[...]
    var = jnp.mean(x * x, axis=1, keepdims=True)
    t = x * jax.lax.rsqrt(var + _EPS) * rw_ref[...]
    tf_ref[...] = t

    logits = jax.lax.dot_general(
        t, gw_ref[...], (((1,), (1,)), ((), ())),
        preferred_element_type=jnp.float32) + gb_ref[...]
    iota_e = jax.lax.broadcasted_iota(jnp.int32, (_T, _E), 1)
    m1 = jnp.max(logits, axis=1, keepdims=True)
    i1 = jnp.min(jnp.where(logits == m1, iota_e, _E), axis=1, keepdims=True)
    l2 = jnp.where(iota_e == i1, -jnp.inf, logits)
    m2 = jnp.max(l2, axis=1, keepdims=True)
    i2 = jnp.min(jnp.where(l2 == m2, iota_e, _E), axis=1, keepdims=True)
    a = jnp.exp(m2 - m1)
    w1c = 1.0 / (1.0 + a)
    w2c = a / (1.0 + a)

    oh0 = (iota_e == i1).astype(jnp.float32)
    oh1 = (iota_e == i2).astype(jnp.float32)
    cnt = oh0 + oh1

    tri = (jax.lax.broadcasted_iota(jnp.int32, (_T, _T), 1) <
           jax.lax.broadcasted_iota(jnp.int32, (_T, _T), 0)).astype(jnp.float32)
    csum = jax.lax.dot_general(tri, cnt, (((1,), (0,)), ((), ())),
                               preferred_element_type=jnp.float32)

    ones_t = jnp.ones((_T, 1), jnp.float32)
    tot = jax.lax.dot_general(cnt, ones_t, (((0,), (0,)), ((), ())),
                              preferred_element_type=jnp.float32)
    nb = jnp.floor((tot + (_B - 1)) / _B)
    lower = (jax.lax.broadcasted_iota(jnp.int32, (_E, _E), 1) <=
             jax.lax.broadcasted_iota(jnp.int32, (_E, _E), 0)).astype(jnp.float32)
    bend = jax.lax.dot_general(lower, nb, (((1,), (0,)), ((), ())),
                               preferred_element_type=jnp.float32)
    bstart = bend - nb

    poff0 = _B * jax.lax.dot_general(oh0, bstart, (((1,), (0,)), ((), ())),
                                     preferred_element_type=jnp.float32)
    poff1 = _B * jax.lax.dot_general(oh1, bstart, (((1,), (0,)), ((), ())),
                                     preferred_element_type=jnp.float32)
    pos0 = poff0 + jnp.sum(oh0 * csum, axis=1, keepdims=True)
    pos1 = poff1 + jnp.sum(oh1 * csum, axis=1, keepdims=True)
    q0 = jnp.floor(pos0 * (1.0 / _B))
    r0 = pos0 - _B * q0
    q1 = jnp.floor(pos1 * (1.0 / _B))
    r1 = pos1 - _B * q1

    iota_q = jax.lax.broadcasted_iota(jnp.int32, (_T, _G), 1)
    iota_r = jax.lax.broadcasted_iota(jnp.int32, (_T, _B), 1)
    mq0 = (iota_q == q0.astype(jnp.int32)).astype(jnp.float32)
    mq1 = (iota_q == q1.astype(jnp.int32)).astype(jnp.float32)
    mr0 = (iota_r == r0.astype(jnp.int32)).astype(jnp.float32)
    mr1 = (iota_r == r1.astype(jnp.int32)).astype(jnp.float32)
    tok = jax.lax.broadcasted_iota(jnp.int32, (_T, 1), 0)
    tok_lo = (tok % 256).astype(jnp.float32)
    tok_hi = (tok // 256).astype(jnp.float32)

    def sc2(lhs, rhs):
        return jax.lax.dot_general(lhs, rhs, (((0,), (0,)), ((), ())),
                                   preferred_element_type=jnp.float32)

    stok_ref[...] = (sc2(mq0, mr0 * tok_lo) + 256.0 * sc2(mq0, mr0 * tok_hi) +
                     sc2(mq1, mr1 * tok_lo) + 256.0 * sc2(mq1, mr1 * tok_hi))
    scoef_ref[...] = sc2(mq0, mr0 * w1c) + sc2(mq1, mr1 * w2c)

    ones_e = jnp.ones((_E, 1), jnp.float32)
    bs_row = jax.lax.dot_general(ones_e, bstart, (((1,), (1,)), ((), ())),
                                 preferred_element_type=jnp.float32)[0:1]
    nb_row = jax.lax.dot_general(ones_e, nb, (((1,), (1,)), ((), ())),
                                 preferred_element_type=jnp.float32)[0:1]
    bsn_ref[:, :_E] = bs_row.astype(jnp.int32)
    bsn_ref[:, _E:] = nb_row.astype(jnp.int32)


_EPG = 2


def _moe_kernel(bsn_sref, stok_ref, scoef_ref, tf_ref, x_ref,
                w1_ref, w3_ref, w2_ref, o_ref):
    g = pl.program_id(0)

    @pl.when(g == 0)
    def _():
        o_ref[...] = x_ref[...]

    iota_tb = jax.lax.broadcasted_iota(jnp.int32, (_T, _B), 0)

    for j in range(_EPG):
        e = g * _EPG + j
        bs = bsn_sref[e]
        nbe = bsn_sref[_E + e]

        def body(i, carry, _j=j, _bs=bs):
            row = pl.ds(_bs + i, 1)
            st = stok_ref[row, :]
            sc = scoef_ref[row, :]
            pt = (iota_tb == st.astype(jnp.int32)).astype(jnp.float32)
            tb = jax.lax.dot_general(pt, tf_ref[...],
                                     (((0,), (0,)), ((), ())),
                                     preferred_element_type=jnp.float32)
            gg = jax.lax.dot_general(tb, w1_ref[_j],
                                     (((1,), (1,)), ((), ())),
                                     preferred_element_type=jnp.float32)
            uu = jax.lax.dot_general(tb, w3_ref[_j],
                                     (((1,), (1,)), ((), ())),
                                     preferred_element_type=jnp.float32)
            h = (gg * jax.lax.logistic(gg)) * uu
            y = jax.lax.dot_general(h, w2_ref[_j],
                                    (((1,), (1,)), ((), ())),
                                    preferred_element_type=jnp.float32)
            contrib = jax.lax.dot_general(pt * sc, y,
                                          (((1,), (0,)), ((), ())),
                                          preferred_element_type=jnp.float32)
            o_ref[...] += contrib
            return carry

        jax.lax.fori_loop(0, nbe, body, 0)


def kernel(x, rms_weight, gate_w, gate_b, w1, w3, w2):
    tf, stok, scoef, bsn = pl.pallas_call(
        _router_sort_kernel,
        out_shape=(
            jax.ShapeDtypeStruct((_T, _H), jnp.float32),
            jax.ShapeDtypeStruct((_G, _B), jnp.float32),
            jax.ShapeDtypeStruct((_G, _B), jnp.float32),
            jax.ShapeDtypeStruct((1, 2 * _E), jnp.int32),
        ),
    )(x, rms_weight.reshape(1, _H), gate_w, gate_b.reshape(1, _E))

    grid_spec = pltpu.PrefetchScalarGridSpec(
        num_scalar_prefetch=1,
        grid=(_E // _EPG,),
        in_specs=[
            pl.BlockSpec((_G, _B), lambda g, s: (0, 0)),
            pl.BlockSpec((_G, _B), lambda g, s: (0, 0)),
            pl.BlockSpec((_T, _H), lambda g, s: (0, 0)),
            pl.BlockSpec((_T, _H), lambda g, s: (0, 0)),
            pl.BlockSpec((_EPG, _DFF, _H), lambda g, s: (g, 0, 0)),
            pl.BlockSpec((_EPG, _DFF, _H), lambda g, s: (g, 0, 0)),
            pl.BlockSpec((_EPG, _H, _DFF), lambda g, s: (g, 0, 0)),
        ],
        out_specs=pl.BlockSpec((_T, _H), lambda g, s: (0, 0)),
    )
    out = pl.pallas_call(
        _moe_kernel,
        grid_spec=grid_spec,
        out_shape=jax.ShapeDtypeStruct((_T, _H), jnp.float32),
        compiler_params=pltpu.CompilerParams(
            dimension_semantics=("arbitrary",)),
    )(bsn.reshape(2 * _E), stok, scoef, tf, x, w1, w3, w2)
    return out


# speedup vs baseline: 1.6361x; 1.0041x over previous
import jax
import jax.numpy as jnp
from jax.experimental import pallas as pl
from jax.experimental.pallas import tpu as pltpu

_T = 512
_H = 768
_DFF = 768
_E = 64
_EPS = 1e-6

_B = 32
_G = 96


def _router_sort_kernel(x_ref, rw_ref, gw_ref, gb_ref,
                        tf_ref, stok_ref, scoef_ref, bsn_ref):
    x = x_ref[...]
    var = jnp.mean(x * x, axis=1, keepdims=True)
    t = x * jax.lax.rsqrt(var + _EPS) * rw_ref[...]
    tf_ref[...] = t

    logits = jax.lax.dot_general(
        t, gw_ref[...], (((1,), (1,)), ((), ())),
        preferred_element_type=jnp.float32) + gb_ref[...]
    iota_e = jax.lax.broadcasted_iota(jnp.int32, (_T, _E), 1)
    m1 = jnp.max(logits, axis=1, keepdims=True)
    i1 = jnp.min(jnp.where(logits == m1, iota_e, _E), axis=1, keepdims=True)
    l2 = jnp.where(iota_e == i1, -jnp.inf, logits)
    m2 = jnp.max(l2, axis=1, keepdims=True)
    i2 = jnp.min(jnp.where(l2 == m2, iota_e, _E), axis=1, keepdims=True)
    a = jnp.exp(m2 - m1)
    w1c = 1.0 / (1.0 + a)
    w2c = a / (1.0 + a)

    oh0 = (iota_e == i1).astype(jnp.float32)
    oh1 = (iota_e == i2).astype(jnp.float32)
    cnt = oh0 + oh1

    tri = (jax.lax.broadcasted_iota(jnp.int32, (_T, _T), 1) <
           jax.lax.broadcasted_iota(jnp.int32, (_T, _T), 0)).astype(jnp.float32)
    csum = jax.lax.dot_general(tri, cnt, (((1,), (0,)), ((), ())),
                               preferred_element_type=jnp.float32)

    ones_t = jnp.ones((_T, 1), jnp.float32)
    tot = jax.lax.dot_general(cnt, ones_t, (((0,), (0,)), ((), ())),
                              preferred_element_type=jnp.float32)
    nb = jnp.floor((tot + (_B - 1)) / _B)
    lower = (jax.lax.broadcasted_iota(jnp.int32, (_E, _E), 1) <=
             jax.lax.broadcasted_iota(jnp.int32, (_E, _E), 0)).astype(jnp.float32)
    bend = jax.lax.dot_general(lower, nb, (((1,), (0,)), ((), ())),
                               preferred_element_type=jnp.float32)
    bstart = bend - nb

    poff0 = _B * jax.lax.dot_general(oh0, bstart, (((1,), (0,)), ((), ())),
                                     preferred_element_type=jnp.float32)
    poff1 = _B * jax.lax.dot_general(oh1, bstart, (((1,), (0,)), ((), ())),
                                     preferred_element_type=jnp.float32)
    pos0 = poff0 + jnp.sum(oh0 * csum, axis=1, keepdims=True)
    pos1 = poff1 + jnp.sum(oh1 * csum, axis=1, keepdims=True)
    q0 = jnp.floor(pos0 * (1.0 / _B))
    r0 = pos0 - _B * q0
    q1 = jnp.floor(pos1 * (1.0 / _B))
    r1 = pos1 - _B * q1

    iota_q = jax.lax.broadcasted_iota(jnp.int32, (_T, _G), 1)
    iota_r = jax.lax.broadcasted_iota(jnp.int32, (_T, _B), 1)
    mq0 = (iota_q == q0.astype(jnp.int32)).astype(jnp.float32)
    mq1 = (iota_q == q1.astype(jnp.int32)).astype(jnp.float32)
    mr0 = (iota_r == r0.astype(jnp.int32)).astype(jnp.float32)
    mr1 = (iota_r == r1.astype(jnp.int32)).astype(jnp.float32)
    tok = jax.lax.broadcasted_iota(jnp.int32, (_T, 1), 0)
    tok_lo = (tok % 256).astype(jnp.float32)
    tok_hi = (tok // 256).astype(jnp.float32)

    def sc2(lhs, rhs):
        return jax.lax.dot_general(lhs, rhs, (((0,), (0,)), ((), ())),
                                   preferred_element_type=jnp.float32)

    stok_ref[...] = (sc2(mq0, mr0 * tok_lo) + 256.0 * sc2(mq0, mr0 * tok_hi) +
                     sc2(mq1, mr1 * tok_lo) + 256.0 * sc2(mq1, mr1 * tok_hi))
    scoef_ref[...] = sc2(mq0, mr0 * w1c) + sc2(mq1, mr1 * w2c)

    ones_e = jnp.ones((_E, 1), jnp.float32)
    bs_row = jax.lax.dot_general(ones_e, bstart, (((1,), (1,)), ((), ())),
                                 preferred_element_type=jnp.float32)[0:1]
    nb_row = jax.lax.dot_general(ones_e, nb, (((1,), (1,)), ((), ())),
                                 preferred_element_type=jnp.float32)[0:1]
    bsn_ref[:, :_E] = bs_row.astype(jnp.int32)
    bsn_ref[:, _E:] = nb_row.astype(jnp.int32)


_EPG = 4


def _moe_kernel(bsn_sref, stok_ref, scoef_ref, tf_ref, x_ref,
                w1_ref, w3_ref, w2_ref, o_ref):
    g = pl.program_id(0)

    @pl.when(g == 0)
    def _():
        o_ref[...] = x_ref[...]

    iota_tb = jax.lax.broadcasted_iota(jnp.int32, (_T, _B), 0)

    for j in range(_EPG):
        e = g * _EPG + j
        bs = bsn_sref[e]
        nbe = bsn_sref[_E + e]

        def body(i, carry, _j=j, _bs=bs):
            row = pl.ds(_bs + i, 1)
            st = stok_ref[row, :]
            sc = scoef_ref[row, :]
            pt = (iota_tb == st.astype(jnp.int32)).astype(jnp.float32)
            tb = jax.lax.dot_general(pt, tf_ref[...],
                                     (((0,), (0,)), ((), ())),
                                     preferred_element_type=jnp.float32)
            gg = jax.lax.dot_general(tb, w1_ref[_j],
                                     (((1,), (1,)), ((), ())),
                                     preferred_element_type=jnp.float32)
            uu = jax.lax.dot_general(tb, w3_ref[_j],
                                     (((1,), (1,)), ((), ())),
                                     preferred_element_type=jnp.float32)
            h = (gg * jax.lax.logistic(gg)) * uu
            y = jax.lax.dot_general(h, w2_ref[_j],
                                    (((1,), (1,)), ((), ())),
                                    preferred_element_type=jnp.float32)
            contrib = jax.lax.dot_general(pt * sc, y,
                                          (((1,), (0,)), ((), ())),
                                          preferred_element_type=jnp.float32)
            o_ref[...] += contrib
            return carry

        jax.lax.fori_loop(0, nbe, body, 0)


def kernel(x, rms_weight, gate_w, gate_b, w1, w3, w2):
    tf, stok, scoef, bsn = pl.pallas_call(
        _router_sort_kernel,
        out_shape=(
            jax.ShapeDtypeStruct((_T, _H), jnp.float32),
            jax.ShapeDtypeStruct((_G, _B), jnp.float32),
            jax.ShapeDtypeStruct((_G, _B), jnp.float32),
            jax.ShapeDtypeStruct((1, 2 * _E), jnp.int32),
        ),
    )(x, rms_weight.reshape(1, _H), gate_w, gate_b.reshape(1, _E))

    grid_spec = pltpu.PrefetchScalarGridSpec(
        num_scalar_prefetch=1,
        grid=(_E // _EPG,),
        in_specs=[
            pl.BlockSpec((_G, _B), lambda g, s: (0, 0)),
            pl.BlockSpec((_G, _B), lambda g, s: (0, 0)),
            pl.BlockSpec((_T, _H), lambda g, s: (0, 0)),
            pl.BlockSpec((_T, _H), lambda g, s: (0, 0)),
            pl.BlockSpec((_EPG, _DFF, _H), lambda g, s: (g, 0, 0)),
            pl.BlockSpec((_EPG, _DFF, _H), lambda g, s: (g, 0, 0)),
            pl.BlockSpec((_EPG, _H, _DFF), lambda g, s: (g, 0, 0)),
        ],
        out_specs=pl.BlockSpec((_T, _H), lambda g, s: (0, 0)),
    )
    out = pl.pallas_call(
        _moe_kernel,
        grid_spec=grid_spec,
        out_shape=jax.ShapeDtypeStruct((_T, _H), jnp.float32),
        compiler_params=pltpu.CompilerParams(
            dimension_semantics=("arbitrary",),
            vmem_limit_bytes=120 * 1024 * 1024),
    )(bsn.reshape(2 * _E), stok, scoef, tf, x, w1, w3, w2)
    return out
